# Initial kernel scaffold; baseline (speedup 1.0000x reference)
import jax
import jax.numpy as jnp
from jax.experimental import pallas as pl


def kernel(text_embeddings, number_percentile_floor, number_percentile_delta, date_year_month_day_weekday, column_embeddings, target, number_table, year_table, month_table, day_table, weekday_table, col_W, col_b, cont_W, cont_b, target_table, ln_gamma, ln_beta):
    raise NotImplementedError("write your pallas kernel here")


# speedup vs baseline: 8.3017x; 8.3017x over previous
import jax
import jax.numpy as jnp
from jax.experimental import pallas as pl
from jax.experimental.pallas import tpu as pltpu

EPS = 1e-12


def _body(te_ref, floor_ref, delta_ref, date_ref, tgt_ref, colemb_ref,
          comb_ref, tgt_tab_ref, colW_ref, colb_ref, contW_ref, contb_ref,
          gamma_ref, beta_ref, out_ref, *, BR, C, EMB, H, Q, NCOMB):
    BRC = BR * C
    te = te_ref[...]
    cidx = jax.lax.broadcasted_iota(jnp.int32, (BR, C, 1), 1)
    te = jnp.where(cidx == C - 1, 0.0, te)
    content = jnp.reshape(te, (BRC, EMB)) @ contW_ref[...] + contb_ref[...]
    colmap = colemb_ref[...] @ colW_ref[...] + colb_ref[...]
    floor = floor_ref[...][:, :, None]
    delta = delta_ref[...][:, :, None]
    mask = floor > -99
    safe = jnp.clip(floor, 0, Q - 1)
    nxt = jnp.minimum(safe + 1, Q - 1)
    q = jax.lax.broadcasted_iota(jnp.int32, (BR, C, NCOMB), 2)
    w = jnp.where(q == safe, 1.0 - delta, 0.0) + jnp.where(q == nxt, delta, 0.0)
    w = jnp.where(mask, w, 0.0)
    d = date_ref[...]
    offs = (Q, Q + 52, Q + 65, Q + 97)
    for j in range(4):
        dj = d[j][:, :, None] + offs[j]
        w = w + jnp.where(q == dj, 1.0, 0.0)
    embeds = jnp.reshape(w, (BRC, NCOMB)) @ comb_ref[...]
    x = jnp.reshape(content + embeds, (BR, C, H)) + colmap[None, :, :]
    tgt = tgt_ref[...]
    t = jnp.where(tgt < 0, 0, tgt + 1)
    qq = jax.lax.broadcasted_iota(jnp.int32, (BR, Q), 1)
    temb = jnp.where(qq == t, 1.0, 0.0) @ tgt_tab_ref[...]
    x = x + jnp.where(cidx == C - 1, temb[:, None, :], 0.0)
    mean = jnp.mean(x, axis=-1, keepdims=True)
    xc = x - mean
    var = jnp.mean(xc * xc, axis=-1, keepdims=True)
    out_ref[...] = xc * jax.lax.rsqrt(var + EPS) * gamma_ref[...] + beta_ref[...]


def kernel(text_embeddings, number_percentile_floor, number_percentile_delta,
           date_year_month_day_weekday, column_embeddings, target,
           number_table, year_table, month_table, day_table, weekday_table,
           col_W, col_b, cont_W, cont_b, target_table, ln_gamma, ln_beta):
    R, C, EMB = text_embeddings.shape
    Q, H = number_table.shape
    BR = 64 if R % 64 == 0 else R
    comb = jnp.concatenate(
        [number_table, year_table, month_table, day_table, weekday_table], axis=0)
    NCOMB = comb.shape[0]
    date_t = jnp.transpose(date_year_month_day_weekday, (2, 0, 1))
    tgt2 = target.reshape(R, 1)
    colb2 = col_b.reshape(1, H)
    contb2 = cont_b.reshape(1, H)
    gamma2 = ln_gamma.reshape(1, 1, H)
    beta2 = ln_beta.reshape(1, 1, H)

    import functools
    body = functools.partial(_body, BR=BR, C=C, EMB=EMB, H=H, Q=Q, NCOMB=NCOMB)
    grid = (R // BR,)
    out = pl.pallas_call(
        body,
        grid=grid,
        in_specs=[
            pl.BlockSpec((BR, C, EMB), lambda i: (i, 0, 0)),
            pl.BlockSpec((BR, C), lambda i: (i, 0)),
            pl.BlockSpec((BR, C), lambda i: (i, 0)),
            pl.BlockSpec((4, BR, C), lambda i: (0, i, 0)),
            pl.BlockSpec((BR, 1), lambda i: (i, 0)),
            pl.BlockSpec((C, EMB), lambda i: (0, 0)),
            pl.BlockSpec((NCOMB, H), lambda i: (0, 0)),
            pl.BlockSpec((Q, H), lambda i: (0, 0)),
            pl.BlockSpec((EMB, H), lambda i: (0, 0)),
            pl.BlockSpec((1, H), lambda i: (0, 0)),
            pl.BlockSpec((EMB, H), lambda i: (0, 0)),
            pl.BlockSpec((1, H), lambda i: (0, 0)),
            pl.BlockSpec((1, 1, H), lambda i: (0, 0, 0)),
            pl.BlockSpec((1, 1, H), lambda i: (0, 0, 0)),
        ],
        out_specs=pl.BlockSpec((BR, C, H), lambda i: (i, 0, 0)),
        out_shape=jax.ShapeDtypeStruct((R, C, H), jnp.float32),
        compiler_params=pltpu.CompilerParams(
            dimension_semantics=("parallel",)),
    )(text_embeddings, number_percentile_floor, number_percentile_delta,
      date_t, tgt2, column_embeddings, comb, target_table,
      col_W, colb2, cont_W, contb2, gamma2, beta2)
    return out
